# baseline (speedup 1.0000x reference)
import functools

import jax
import jax.numpy as jnp
from jax.experimental import pallas as pl
from jax.experimental.pallas import tpu as pltpu

N = 4096
NH = 512
ALPHA = 0.2
NEG = -9e15

BM = 256
BN = 1024
BMM = 512


def _mm_kernel(h_ref, w_ref, o_ref, *, relu):
    h = h_ref[...]
    if relu:
        h = jnp.maximum(h, 0.0)
    o_ref[...] = jnp.dot(h, w_ref[...], preferred_element_type=jnp.float32)


def _mm(h, w, relu):
    n, nin = h.shape
    nh = w.shape[1]
    return pl.pallas_call(
        functools.partial(_mm_kernel, relu=relu),
        grid=(n // BMM,),
        in_specs=[
            pl.BlockSpec((BMM, nin), lambda i: (0, 0) if n == BMM else (i, 0)),
            pl.BlockSpec((nin, nh), lambda i: (0, 0)),
        ],
        out_specs=pl.BlockSpec((BMM, nh), lambda i: (0, 0) if n == BMM else (i, 0)),
        out_shape=jax.ShapeDtypeStruct((n, nh), jnp.float32),
        compiler_params=pltpu.CompilerParams(
            dimension_semantics=("parallel",),
        ),
    )(h, w)


def _attn_kernel(adj_ref, wh_ref, asrc_ref, adst_ref, o_ref,
                 acc_ref, m_ref, l_ref, *, nj):
    i = pl.program_id(0)
    j = pl.program_id(1)

    wh_i = wh_ref[pl.ds(i * BM, BM), :]
    wh_j = wh_ref[pl.ds(j * BN, BN), :]
    s = jnp.dot(wh_i, asrc_ref[...],
                preferred_element_type=jnp.float32)
    d = jnp.dot(wh_j, adst_ref[...],
                preferred_element_type=jnp.float32)

    e = s + d.T
    e = jnp.where(e >= 0.0, e, ALPHA * e)
    e = jnp.where(adj_ref[...] > 0.99, e, NEG)

    m_blk = jnp.max(e, axis=1, keepdims=True)

    @pl.when(j == 0)
    def _init():
        p = jnp.exp(e - m_blk)
        m_ref[...] = m_blk
        l_ref[...] = jnp.sum(p, axis=1, keepdims=True)
        acc_ref[...] = jnp.dot(p, wh_j, preferred_element_type=jnp.float32)

    @pl.when(j > 0)
    def _step():
        m_old = m_ref[...]
        m_new = jnp.maximum(m_old, m_blk)
        p = jnp.exp(e - m_new)
        scale = jnp.exp(m_old - m_new)
        m_ref[...] = m_new
        l_ref[...] = l_ref[...] * scale + jnp.sum(p, axis=1, keepdims=True)
        acc_ref[...] = acc_ref[...] * scale + jnp.dot(
            p, wh_j, preferred_element_type=jnp.float32)

    @pl.when(j == nj - 1)
    def _finish():
        o_ref[...] = acc_ref[...] / l_ref[...]


def _attn(adj, wh, a):
    n, nh = wh.shape
    a_src = a[:nh]
    a_dst = a[nh:]
    ni = n // BM
    nj = n // BN
    return pl.pallas_call(
        functools.partial(_attn_kernel, nj=nj),
        grid=(ni, nj),
        in_specs=[
            pl.BlockSpec((BM, BN), lambda i, j: (i, j)),
            pl.BlockSpec((n, nh), lambda i, j: (0, 0)),
            pl.BlockSpec((nh, 1), lambda i, j: (0, 0)),
            pl.BlockSpec((nh, 1), lambda i, j: (0, 0)),
        ],
        out_specs=pl.BlockSpec((BM, nh), lambda i, j: (i, 0)),
        out_shape=jax.ShapeDtypeStruct((n, nh), jnp.float32),
        scratch_shapes=[
            pltpu.VMEM((BM, nh), jnp.float32),
            pltpu.VMEM((BM, 1), jnp.float32),
            pltpu.VMEM((BM, 1), jnp.float32),
        ],
        compiler_params=pltpu.CompilerParams(
            dimension_semantics=("parallel", "arbitrary"),
        ),
    )(adj, wh, a_src, a_dst)


def kernel(features, adj_matrix, W1, a1, W2, a2, W3, a3):
    wh = _mm(features, W1, relu=False)
    h = _attn(adj_matrix, wh, a1)
    wh = _mm(h, W2, relu=True)
    h = _attn(adj_matrix, wh, a2)
    wh = _mm(h, W3, relu=True)
    h = _attn(adj_matrix, wh, a3)
    return h


# speedup vs baseline: 1.7024x; 1.7024x over previous
import functools

import jax
import jax.numpy as jnp
from jax.experimental import pallas as pl
from jax.experimental.pallas import tpu as pltpu

ALPHA = 0.2
NHA = 640

BM = 512
BN = 2048
BMM = 512


def _mm_kernel(h_ref, w_ref, asrc_ref, adst_ref,
               wh_ref, s_ref, dt_ref, maxd_ref, *, relu):
    i = pl.program_id(0)
    h = h_ref[...]
    if relu:
        h = jnp.maximum(h, 0.0)
    wh = jnp.dot(h, w_ref[...], preferred_element_type=jnp.float32)
    s_ref[...] = jnp.dot(wh, asrc_ref[...], preferred_element_type=jnp.float32)
    d = jnp.dot(wh, adst_ref[...], preferred_element_type=jnp.float32)
    dt_ref[...] = d.T
    wh_ref[:, :512] = wh.astype(jnp.bfloat16)
    lane = jax.lax.broadcasted_iota(jnp.int32, (BMM, NHA - 512), 1)
    wh_ref[:, 512:] = jnp.where(lane == 0, 1.0, 0.0).astype(jnp.bfloat16)
    local_max = jnp.max(d, axis=0, keepdims=True)

    @pl.when(i == 0)
    def _first():
        maxd_ref[...] = local_max

    @pl.when(i > 0)
    def _rest():
        maxd_ref[...] = jnp.maximum(maxd_ref[...], local_max)


def _mm(h, w, a, relu):
    n, nin = h.shape
    nh = w.shape[1]
    a_src = a[:nh]
    a_dst = a[nh:]
    return pl.pallas_call(
        functools.partial(_mm_kernel, relu=relu),
        grid=(n // BMM,),
        in_specs=[
            pl.BlockSpec((BMM, nin), lambda i: (i, 0)),
            pl.BlockSpec((nin, nh), lambda i: (0, 0)),
            pl.BlockSpec((nh, 1), lambda i: (0, 0)),
            pl.BlockSpec((nh, 1), lambda i: (0, 0)),
        ],
        out_specs=[
            pl.BlockSpec((BMM, NHA), lambda i: (i, 0)),
            pl.BlockSpec((BMM, 1), lambda i: (i, 0)),
            pl.BlockSpec((1, BMM), lambda i: (0, i)),
            pl.BlockSpec((1, 1), lambda i: (0, 0)),
        ],
        out_shape=[
            jax.ShapeDtypeStruct((n, NHA), jnp.bfloat16),
            jax.ShapeDtypeStruct((n, 1), jnp.float32),
            jax.ShapeDtypeStruct((1, n), jnp.float32),
            jax.ShapeDtypeStruct((1, 1), jnp.float32),
        ],
        compiler_params=pltpu.CompilerParams(
            dimension_semantics=("arbitrary",),
        ),
    )(h, w, a_src, a_dst)


def _attn_body(masked, wh_ref, s_ref, dt_ref, maxd_ref, o_ref,
               acc_ref, ab_ref, *, nj):
    j = pl.program_id(1)

    @pl.when(j == 0)
    def _init():
        s = s_ref[...]
        x = s + maxd_ref[...]
        m = jnp.maximum(x, ALPHA * x)
        ab_ref[:, 0:1] = s - m
        ab_ref[:, 1:2] = ALPHA * s - m

    a = ab_ref[:, 0:1]
    b = ab_ref[:, 1:2]
    dt = dt_ref[...]
    t = jnp.maximum(a + dt, b + ALPHA * dt)
    p16 = jnp.exp(t).astype(jnp.bfloat16) * masked
    wh_j = wh_ref[pl.ds(j * BN, BN), :]
    pv = jnp.dot(p16, wh_j, preferred_element_type=jnp.float32)

    @pl.when(j == 0)
    def _acc0():
        acc_ref[...] = pv

    @pl.when(j > 0)
    def _acc():
        acc_ref[...] += pv

    @pl.when(j == nj - 1)
    def _finish():
        o_ref[...] = acc_ref[:, :512] / acc_ref[:, 512:513]


def _attn1_kernel(adj_ref, wh_ref, s_ref, dt_ref, maxd_ref,
                  o_ref, mask_ref, acc_ref, ab_ref, *, nj):
    mask16 = (adj_ref[...] > 0.99).astype(jnp.bfloat16)
    mask_ref[...] = mask16
    _attn_body(mask16, wh_ref, s_ref, dt_ref, maxd_ref, o_ref,
               acc_ref, ab_ref, nj=nj)


def _attn23_kernel(mask_ref, wh_ref, s_ref, dt_ref, maxd_ref,
                   o_ref, acc_ref, ab_ref, *, nj):
    _attn_body(mask_ref[...], wh_ref, s_ref, dt_ref, maxd_ref, o_ref,
               acc_ref, ab_ref, nj=nj)


def _attn(mat, wh, s, dt, maxd, first):
    n = s.shape[0]
    ni = n // BM
    nj = n // BN
    in_specs = [
        pl.BlockSpec((BM, BN), lambda i, j: (i, j)),
        pl.BlockSpec((n, NHA), lambda i, j: (0, 0)),
        pl.BlockSpec((BM, 1), lambda i, j: (i, 0)),
        pl.BlockSpec((1, BN), lambda i, j: (0, j)),
        pl.BlockSpec((1, 1), lambda i, j: (0, 0)),
    ]
    out_shape = [jax.ShapeDtypeStruct((n, 512), jnp.float32)]
    out_specs = [pl.BlockSpec((BM, 512), lambda i, j: (i, 0))]
    if first:
        body = _attn1_kernel
        out_shape.append(jax.ShapeDtypeStruct((n, n), jnp.bfloat16))
        out_specs.append(pl.BlockSpec((BM, BN), lambda i, j: (i, j)))
    else:
        body = _attn23_kernel
    res = pl.pallas_call(
        functools.partial(body, nj=nj),
        grid=(ni, nj),
        in_specs=in_specs,
        out_specs=out_specs,
        out_shape=out_shape,
        scratch_shapes=[
            pltpu.VMEM((BM, NHA), jnp.float32),
            pltpu.VMEM((BM, 2), jnp.float32),
        ],
        compiler_params=pltpu.CompilerParams(
            dimension_semantics=("parallel", "arbitrary"),
        ),
    )(mat, wh, s, dt, maxd)
    if first:
        return res[0], res[1]
    return res[0]


def kernel(features, adj_matrix, W1, a1, W2, a2, W3, a3):
    wh, s, dt, maxd = _mm(features, W1, a1, relu=False)
    h, mask = _attn(adj_matrix, wh, s, dt, maxd, first=True)
    wh, s, dt, maxd = _mm(h, W2, a2, relu=True)
    h = _attn(mask, wh, s, dt, maxd, first=False)
    wh, s, dt, maxd = _mm(h, W3, a3, relu=True)
    h = _attn(mask, wh, s, dt, maxd, first=False)
    return h


# speedup vs baseline: 2.0190x; 1.1860x over previous
import functools

import jax
import jax.numpy as jnp
from jax.experimental import pallas as pl
from jax.experimental.pallas import tpu as pltpu

ALPHA = 0.2
LOG2E = 1.4426950408889634
NHA = 640

BM = 512
BMM = 512


def _mm_kernel(h_ref, w_ref, asrc_ref, adst_ref,
               wh_ref, s_ref, dt_ref, maxd_ref, *, relu):
    i = pl.program_id(0)
    h = h_ref[...]
    if relu:
        h = jnp.maximum(h, 0.0)
    wh = jnp.dot(h, w_ref[...], preferred_element_type=jnp.float32)
    s_ref[...] = jnp.dot(wh, asrc_ref[...], preferred_element_type=jnp.float32)
    d = jnp.dot(wh, adst_ref[...], preferred_element_type=jnp.float32)
    dt_ref[...] = d.T
    wh_ref[:, :512] = wh.astype(jnp.bfloat16)
    lane = jax.lax.broadcasted_iota(jnp.int32, (BMM, NHA - 512), 1)
    wh_ref[:, 512:] = jnp.where(lane == 0, 1.0, 0.0).astype(jnp.bfloat16)
    local_max = jnp.max(d, axis=0, keepdims=True)

    @pl.when(i == 0)
    def _first():
        maxd_ref[...] = local_max

    @pl.when(i > 0)
    def _rest():
        maxd_ref[...] = jnp.maximum(maxd_ref[...], local_max)


def _mm(h, w, a, relu):
    n, nin = h.shape
    nh = w.shape[1]
    a2 = a * LOG2E
    a_src = a2[:nh]
    a_dst = a2[nh:]
    return pl.pallas_call(
        functools.partial(_mm_kernel, relu=relu),
        grid=(n // BMM,),
        in_specs=[
            pl.BlockSpec((BMM, nin), lambda i: (i, 0)),
            pl.BlockSpec((nin, nh), lambda i: (0, 0)),
            pl.BlockSpec((nh, 1), lambda i: (0, 0)),
            pl.BlockSpec((nh, 1), lambda i: (0, 0)),
        ],
        out_specs=[
            pl.BlockSpec((BMM, NHA), lambda i: (i, 0)),
            pl.BlockSpec((BMM, 1), lambda i: (i, 0)),
            pl.BlockSpec((1, BMM), lambda i: (0, i)),
            pl.BlockSpec((1, 1), lambda i: (0, 0)),
        ],
        out_shape=[
            jax.ShapeDtypeStruct((n, NHA), jnp.bfloat16),
            jax.ShapeDtypeStruct((n, 1), jnp.float32),
            jax.ShapeDtypeStruct((1, n), jnp.float32),
            jax.ShapeDtypeStruct((1, 1), jnp.float32),
        ],
        compiler_params=pltpu.CompilerParams(
            dimension_semantics=("arbitrary",),
        ),
    )(h, w, a_src, a_dst)


def _attn_body(mask16, wh_ref, s_ref, dt_ref, maxd_ref, o_ref):
    s = s_ref[...]
    x = s + maxd_ref[...]
    m = jnp.maximum(x, ALPHA * x)
    a = s - m
    b = ALPHA * s - m
    dt = dt_ref[...]
    t = jnp.maximum(a + dt, b + ALPHA * dt)
    p16 = jnp.exp2(t).astype(jnp.bfloat16) * mask16
    pv = jnp.dot(p16, wh_ref[...], preferred_element_type=jnp.float32)
    o_ref[...] = pv[:, :512] / pv[:, 512:513]


def _attn1_kernel(adj_ref, wh_ref, s_ref, dt_ref, maxd_ref,
                  o_ref, mask_ref):
    mask16 = (adj_ref[...] > 0.99).astype(jnp.bfloat16)
    mask_ref[...] = mask16
    _attn_body(mask16, wh_ref, s_ref, dt_ref, maxd_ref, o_ref)


def _attn23_kernel(mask_ref, wh_ref, s_ref, dt_ref, maxd_ref, o_ref):
    _attn_body(mask_ref[...], wh_ref, s_ref, dt_ref, maxd_ref, o_ref)


def _attn(mat, wh, s, dt, maxd, first):
    n = s.shape[0]
    ni = n // BM
    in_specs = [
        pl.BlockSpec((BM, n), lambda i: (i, 0)),
        pl.BlockSpec((n, NHA), lambda i: (0, 0)),
        pl.BlockSpec((BM, 1), lambda i: (i, 0)),
        pl.BlockSpec((1, n), lambda i: (0, 0)),
        pl.BlockSpec((1, 1), lambda i: (0, 0)),
    ]
    out_shape = [jax.ShapeDtypeStruct((n, 512), jnp.float32)]
    out_specs = [pl.BlockSpec((BM, 512), lambda i: (i, 0))]
    if first:
        body = _attn1_kernel
        out_shape.append(jax.ShapeDtypeStruct((n, n), jnp.bfloat16))
        out_specs.append(pl.BlockSpec((BM, n), lambda i: (i, 0)))
    else:
        body = _attn23_kernel
    res = pl.pallas_call(
        body,
        grid=(ni,),
        in_specs=in_specs,
        out_specs=out_specs,
        out_shape=out_shape,
        compiler_params=pltpu.CompilerParams(
            dimension_semantics=("parallel",),
        ),
    )(mat, wh, s, dt, maxd)
    if first:
        return res[0], res[1]
    return res[0]


def kernel(features, adj_matrix, W1, a1, W2, a2, W3, a3):
    wh, s, dt, maxd = _mm(features, W1, a1, relu=False)
    h, mask = _attn(adj_matrix, wh, s, dt, maxd, first=True)
    wh, s, dt, maxd = _mm(h, W2, a2, relu=True)
    h = _attn(mask, wh, s, dt, maxd, first=False)
    wh, s, dt, maxd = _mm(h, W3, a3, relu=True)
    h = _attn(mask, wh, s, dt, maxd, first=False)
    return h


# speedup vs baseline: 2.0618x; 1.0212x over previous
import functools

import jax
import jax.numpy as jnp
from jax.experimental import pallas as pl
from jax.experimental.pallas import tpu as pltpu

ALPHA = 0.2
LOG2E = 1.4426950408889634
NHA = 640

BM = 512
BMM = 512


def _wh_outputs(i, wh, asd_ref, wh_ref, s_ref, dt_ref, maxd_ref):
    sd = jnp.dot(wh, asd_ref[...], preferred_element_type=jnp.float32)
    s_ref[...] = sd[:, 0:1]
    d = sd[:, 1:2]
    dt_ref[...] = d.T
    wh_ref[:, :512] = wh.astype(jnp.bfloat16)
    lane = jax.lax.broadcasted_iota(jnp.int32, (BMM, NHA - 512), 1)
    wh_ref[:, 512:] = jnp.where(lane == 0, 1.0, 0.0).astype(jnp.bfloat16)
    local_max = jnp.max(d, axis=0, keepdims=True)

    @pl.when(i == 0)
    def _first():
        maxd_ref[...] = local_max

    @pl.when(i > 0)
    def _rest():
        maxd_ref[...] = jnp.maximum(maxd_ref[...], local_max)


def _mm_kernel(h_ref, w_ref, asd_ref, wh_ref, s_ref, dt_ref, maxd_ref):
    i = pl.program_id(0)
    wh = jnp.dot(h_ref[...], w_ref[...], preferred_element_type=jnp.float32)
    _wh_outputs(i, wh, asd_ref, wh_ref, s_ref, dt_ref, maxd_ref)


def _wh_specs(n):
    out_specs = [
        pl.BlockSpec((BMM, NHA), lambda i: (i, 0)),
        pl.BlockSpec((BMM, 1), lambda i: (i, 0)),
        pl.BlockSpec((1, BMM), lambda i: (0, i)),
        pl.BlockSpec((1, 1), lambda i: (0, 0)),
    ]
    out_shape = [
        jax.ShapeDtypeStruct((n, NHA), jnp.bfloat16),
        jax.ShapeDtypeStruct((n, 1), jnp.float32),
        jax.ShapeDtypeStruct((1, n), jnp.float32),
        jax.ShapeDtypeStruct((1, 1), jnp.float32),
    ]
    return out_specs, out_shape


def _asd(a):
    nh = a.shape[0] // 2
    return jnp.concatenate([a[:nh], a[nh:]], axis=1) * LOG2E


def _mm(h, w, a):
    n, nin = h.shape
    out_specs, out_shape = _wh_specs(n)
    return pl.pallas_call(
        _mm_kernel,
        grid=(n // BMM,),
        in_specs=[
            pl.BlockSpec((BMM, nin), lambda i: (i, 0)),
            pl.BlockSpec((nin, w.shape[1]), lambda i: (0, 0)),
            pl.BlockSpec((w.shape[1], 2), lambda i: (0, 0)),
        ],
        out_specs=out_specs,
        out_shape=out_shape,
        compiler_params=pltpu.CompilerParams(
            dimension_semantics=("arbitrary",),
        ),
    )(h, w, _asd(a))


def _attn_out(mask16, wh_ref, s_ref, dt_ref, maxd_ref):
    s = s_ref[...]
    x = s + maxd_ref[...]
    m = jnp.maximum(x, ALPHA * x)
    a = s - m
    b = ALPHA * s - m
    dt = dt_ref[...]
    t = jnp.maximum(a + dt, b + ALPHA * dt)
    p16 = jnp.exp2(t).astype(jnp.bfloat16) * mask16
    pv = jnp.dot(p16, wh_ref[...], preferred_element_type=jnp.float32)
    return pv[:, :512] / pv[:, 512:513]


def _attn1_kernel(adj_ref, wh_ref, s_ref, dt_ref, maxd_ref, wn_ref, asd_ref,
                  mask_ref, who_ref, so_ref, dto_ref, maxdo_ref):
    i = pl.program_id(0)
    mask16 = (adj_ref[...] > 0.99).astype(jnp.bfloat16)
    mask_ref[...] = mask16
    h = _attn_out(mask16, wh_ref, s_ref, dt_ref, maxd_ref)
    wh = jnp.dot(jnp.maximum(h, 0.0), wn_ref[...],
                 preferred_element_type=jnp.float32)
    _wh_outputs(i, wh, asd_ref, who_ref, so_ref, dto_ref, maxdo_ref)


def _attn2_kernel(mask_ref, wh_ref, s_ref, dt_ref, maxd_ref, wn_ref, asd_ref,
                  who_ref, so_ref, dto_ref, maxdo_ref):
    i = pl.program_id(0)
    h = _attn_out(mask_ref[...], wh_ref, s_ref, dt_ref, maxd_ref)
    wh = jnp.dot(jnp.maximum(h, 0.0), wn_ref[...],
                 preferred_element_type=jnp.float32)
    _wh_outputs(i, wh, asd_ref, who_ref, so_ref, dto_ref, maxdo_ref)


def _attn3_kernel(mask_ref, wh_ref, s_ref, dt_ref, maxd_ref, o_ref):
    o_ref[...] = _attn_out(mask_ref[...], wh_ref, s_ref, dt_ref, maxd_ref)


def _attn_specs(n, first):
    mat_dtype = jnp.float32 if first else jnp.bfloat16
    in_specs = [
        pl.BlockSpec((BM, n), lambda i: (i, 0)),
        pl.BlockSpec((n, NHA), lambda i: (0, 0)),
        pl.BlockSpec((BM, 1), lambda i: (i, 0)),
        pl.BlockSpec((1, n), lambda i: (0, 0)),
        pl.BlockSpec((1, 1), lambda i: (0, 0)),
    ]
    return mat_dtype, in_specs


def _attn_mid(mat, wh, s, dt, maxd, w_next, a_next, first):
    n = s.shape[0]
    _, in_specs = _attn_specs(n, first)
    in_specs += [
        pl.BlockSpec((512, 512), lambda i: (0, 0)),
        pl.BlockSpec((512, 2), lambda i: (0, 0)),
    ]
    out_specs, out_shape = _wh_specs(n)
    if first:
        body = _attn1_kernel
        out_specs = [pl.BlockSpec((BM, n), lambda i: (i, 0))] + out_specs
        out_shape = [jax.ShapeDtypeStruct((n, n), jnp.bfloat16)] + out_shape
    else:
        body = _attn2_kernel
    return pl.pallas_call(
        body,
        grid=(n // BM,),
        in_specs=in_specs,
        out_specs=out_specs,
        out_shape=out_shape,
        compiler_params=pltpu.CompilerParams(
            dimension_semantics=("arbitrary",),
        ),
    )(mat, wh, s, dt, maxd, w_next, _asd(a_next))


def _attn_last(mask, wh, s, dt, maxd):
    n = s.shape[0]
    _, in_specs = _attn_specs(n, False)
    return pl.pallas_call(
        _attn3_kernel,
        grid=(n // BM,),
        in_specs=in_specs,
        out_specs=pl.BlockSpec((BM, 512), lambda i: (i, 0)),
        out_shape=jax.ShapeDtypeStruct((n, 512), jnp.float32),
        compiler_params=pltpu.CompilerParams(
            dimension_semantics=("parallel",),
        ),
    )(mask, wh, s, dt, maxd)


def kernel(features, adj_matrix, W1, a1, W2, a2, W3, a3):
    wh, s, dt, maxd = _mm(features, W1, a1)
    mask, wh, s, dt, maxd = _attn_mid(adj_matrix, wh, s, dt, maxd,
                                      W2, a2, first=True)
    wh, s, dt, maxd = _attn_mid(mask, wh, s, dt, maxd, W3, a3, first=False)
    return _attn_last(mask, wh, s, dt, maxd)


# speedup vs baseline: 2.1111x; 1.0239x over previous
import functools

import jax
import jax.numpy as jnp
from jax.experimental import pallas as pl
from jax.experimental.pallas import tpu as pltpu

ALPHA = 0.2
LOG2E = 1.4426950408889634
NHA = 640

BM = 512
BMM = 512


def _wh_outputs(i, wh, asd_ref, wh_ref, s_ref, dt_ref, maxd_ref):
    sd = jnp.dot(wh, asd_ref[...], preferred_element_type=jnp.float32)
    s_ref[...] = sd[:, 0:1]
    d = sd[:, 1:2]
    dt_ref[...] = d.T
    wh_ref[:, :512] = wh.astype(jnp.bfloat16)
    lane = jax.lax.broadcasted_iota(jnp.int32, (BMM, NHA - 512), 1)
    wh_ref[:, 512:] = jnp.where(lane == 0, 1.0, 0.0).astype(jnp.bfloat16)
    local_max = jnp.max(d, axis=0, keepdims=True)

    @pl.when(i == 0)
    def _first():
        maxd_ref[...] = local_max

    @pl.when(i > 0)
    def _rest():
        maxd_ref[...] = jnp.maximum(maxd_ref[...], local_max)


def _mm_kernel(h_ref, w_ref, asd_ref, wh_ref, s_ref, dt_ref, maxd_ref):
    i = pl.program_id(0)
    wh = jnp.dot(h_ref[...], w_ref[...], preferred_element_type=jnp.float32)
    _wh_outputs(i, wh, asd_ref, wh_ref, s_ref, dt_ref, maxd_ref)


def _wh_specs(n):
    out_specs = [
        pl.BlockSpec((BMM, NHA), lambda i: (i, 0)),
        pl.BlockSpec((BMM, 1), lambda i: (i, 0)),
        pl.BlockSpec((1, BMM), lambda i: (0, i)),
        pl.BlockSpec((1, 1), lambda i: (0, 0)),
    ]
    out_shape = [
        jax.ShapeDtypeStruct((n, NHA), jnp.bfloat16),
        jax.ShapeDtypeStruct((n, 1), jnp.float32),
        jax.ShapeDtypeStruct((1, n), jnp.float32),
        jax.ShapeDtypeStruct((1, 1), jnp.float32),
    ]
    return out_specs, out_shape


def _asd(a):
    nh = a.shape[0] // 2
    return jnp.concatenate([a[:nh], a[nh:]], axis=1) * LOG2E


def _mm(h, w, a):
    n, nin = h.shape
    out_specs, out_shape = _wh_specs(n)
    return pl.pallas_call(
        _mm_kernel,
        grid=(n // BMM,),
        in_specs=[
            pl.BlockSpec((BMM, nin), lambda i: (i, 0)),
            pl.BlockSpec((nin, w.shape[1]), lambda i: (0, 0)),
            pl.BlockSpec((w.shape[1], 2), lambda i: (0, 0)),
        ],
        out_specs=out_specs,
        out_shape=out_shape,
        compiler_params=pltpu.CompilerParams(
            dimension_semantics=("arbitrary",),
        ),
    )(h, w, _asd(a))


def _attn_out(mask16, wh_ref, s_ref, dt_ref, maxd_ref):
    s = s_ref[...]
    x = s + maxd_ref[...]
    m = jnp.maximum(x, ALPHA * x)
    a = s - m
    b = ALPHA * s - m
    dt = dt_ref[...]
    t = jnp.maximum(a + dt, b + ALPHA * dt)
    p16 = jnp.exp2(t).astype(jnp.bfloat16) * mask16
    pv = jnp.dot(p16, wh_ref[...], preferred_element_type=jnp.float32)
    return pv[:, :512] / pv[:, 512:513]


def _attn1_kernel(adj_ref, wh_ref, s_ref, dt_ref, maxd_ref, wn_ref, asd_ref,
                  mask_ref, who_ref, so_ref, dto_ref, maxdo_ref):
    i = pl.program_id(0)
    masked = adj_ref[...] > 0.99
    mask_ref[...] = masked.astype(jnp.int8)
    h = _attn_out(masked.astype(jnp.bfloat16), wh_ref, s_ref, dt_ref,
                  maxd_ref)
    wh = jnp.dot(jnp.maximum(h, 0.0), wn_ref[...],
                 preferred_element_type=jnp.float32)
    _wh_outputs(i, wh, asd_ref, who_ref, so_ref, dto_ref, maxdo_ref)


def _attn2_kernel(mask_ref, wh_ref, s_ref, dt_ref, maxd_ref, wn_ref, asd_ref,
                  who_ref, so_ref, dto_ref, maxdo_ref):
    i = pl.program_id(0)
    h = _attn_out(mask_ref[...].astype(jnp.bfloat16), wh_ref, s_ref, dt_ref,
                  maxd_ref)
    wh = jnp.dot(jnp.maximum(h, 0.0), wn_ref[...],
                 preferred_element_type=jnp.float32)
    _wh_outputs(i, wh, asd_ref, who_ref, so_ref, dto_ref, maxdo_ref)


def _attn3_kernel(mask_ref, wh_ref, s_ref, dt_ref, maxd_ref, o_ref):
    o_ref[...] = _attn_out(mask_ref[...].astype(jnp.bfloat16), wh_ref, s_ref,
                           dt_ref, maxd_ref)


def _attn_specs(n, first):
    mat_dtype = jnp.float32 if first else jnp.bfloat16
    in_specs = [
        pl.BlockSpec((BM, n), lambda i: (i, 0)),
        pl.BlockSpec((n, NHA), lambda i: (0, 0)),
        pl.BlockSpec((BM, 1), lambda i: (i, 0)),
        pl.BlockSpec((1, n), lambda i: (0, 0)),
        pl.BlockSpec((1, 1), lambda i: (0, 0)),
    ]
    return mat_dtype, in_specs


def _attn_mid(mat, wh, s, dt, maxd, w_next, a_next, first):
    n = s.shape[0]
    _, in_specs = _attn_specs(n, first)
    in_specs += [
        pl.BlockSpec((512, 512), lambda i: (0, 0)),
        pl.BlockSpec((512, 2), lambda i: (0, 0)),
    ]
    out_specs, out_shape = _wh_specs(n)
    if first:
        body = _attn1_kernel
        out_specs = [pl.BlockSpec((BM, n), lambda i: (i, 0))] + out_specs
        out_shape = [jax.ShapeDtypeStruct((n, n), jnp.int8)] + out_shape
    else:
        body = _attn2_kernel
    return pl.pallas_call(
        body,
        grid=(n // BM,),
        in_specs=in_specs,
        out_specs=out_specs,
        out_shape=out_shape,
        compiler_params=pltpu.CompilerParams(
            dimension_semantics=("arbitrary",),
        ),
    )(mat, wh, s, dt, maxd, w_next, _asd(a_next))


def _attn_last(mask, wh, s, dt, maxd):
    n = s.shape[0]
    _, in_specs = _attn_specs(n, False)
    return pl.pallas_call(
        _attn3_kernel,
        grid=(n // BM,),
        in_specs=in_specs,
        out_specs=pl.BlockSpec((BM, 512), lambda i: (i, 0)),
        out_shape=jax.ShapeDtypeStruct((n, 512), jnp.float32),
        compiler_params=pltpu.CompilerParams(
            dimension_semantics=("parallel",),
        ),
    )(mask, wh, s, dt, maxd)


def kernel(features, adj_matrix, W1, a1, W2, a2, W3, a3):
    wh, s, dt, maxd = _mm(features, W1, a1)
    mask, wh, s, dt, maxd = _attn_mid(adj_matrix, wh, s, dt, maxd,
                                      W2, a2, first=True)
    wh, s, dt, maxd = _attn_mid(mask, wh, s, dt, maxd, W3, a3, first=False)
    return _attn_last(mask, wh, s, dt, maxd)


# speedup vs baseline: 2.1995x; 1.0419x over previous
import functools

import jax
import jax.numpy as jnp
from jax.experimental import pallas as pl
from jax.experimental.pallas import tpu as pltpu

ALPHA = 0.2
LOG2E = 1.4426950408889634
NHA = 640

BM = 512
BMM = 512


def _wh_outputs(i, wh, asd_ref, wh_ref, s_ref, dt_ref, maxd_ref):
    sd = jnp.dot(wh, asd_ref[...], preferred_element_type=jnp.float32)
    s_ref[...] = sd[:, 0:1]
    d = sd[:, 1:2]
    dt_ref[...] = d.T
    wh_ref[:, :512] = wh.astype(jnp.bfloat16)
    lane = jax.lax.broadcasted_iota(jnp.int32, (BMM, NHA - 512), 1)
    wh_ref[:, 512:] = jnp.where(lane == 0, 1.0, 0.0).astype(jnp.bfloat16)
    local_max = jnp.max(d, axis=0, keepdims=True)

    @pl.when(i == 0)
    def _first():
        maxd_ref[...] = local_max

    @pl.when(i > 0)
    def _rest():
        maxd_ref[...] = jnp.maximum(maxd_ref[...], local_max)


def _mm_kernel(h_ref, w_ref, asd_ref, wh_ref, s_ref, dt_ref, maxd_ref):
    i = pl.program_id(0)
    wh = jnp.dot(h_ref[...], w_ref[...], preferred_element_type=jnp.float32)
    _wh_outputs(i, wh, asd_ref, wh_ref, s_ref, dt_ref, maxd_ref)


def _wh_specs(n):
    out_specs = [
        pl.BlockSpec((BMM, NHA), lambda i: (i, 0)),
        pl.BlockSpec((BMM, 1), lambda i: (i, 0)),
        pl.BlockSpec((1, BMM), lambda i: (0, i)),
        pl.BlockSpec((1, 1), lambda i: (0, 0)),
    ]
    out_shape = [
        jax.ShapeDtypeStruct((n, NHA), jnp.bfloat16),
        jax.ShapeDtypeStruct((n, 1), jnp.float32),
        jax.ShapeDtypeStruct((1, n), jnp.float32),
        jax.ShapeDtypeStruct((1, 1), jnp.float32),
    ]
    return out_specs, out_shape


def _asd(a):
    nh = a.shape[0] // 2
    return jnp.concatenate([a[:nh], a[nh:]], axis=1) * LOG2E


def _mm(h, w, a):
    n, nin = h.shape
    out_specs, out_shape = _wh_specs(n)
    return pl.pallas_call(
        _mm_kernel,
        grid=(n // BMM,),
        in_specs=[
            pl.BlockSpec((BMM, nin), lambda i: (i, 0)),
            pl.BlockSpec((nin, w.shape[1]), lambda i: (0, 0)),
            pl.BlockSpec((w.shape[1], 2), lambda i: (0, 0)),
        ],
        out_specs=out_specs,
        out_shape=out_shape,
        compiler_params=pltpu.CompilerParams(
            dimension_semantics=("arbitrary",),
        ),
    )(h, w, _asd(a))


KC = 1024


def _attn_out(mask_chunk, wh_ref, s_ref, dt_ref, maxd_ref):
    s = s_ref[...]
    x = s + maxd_ref[...]
    m = jnp.maximum(x, ALPHA * x)
    a = s - m
    b = ALPHA * s - m
    dt = dt_ref[...]
    n = dt.shape[1]
    pv = None
    for c in range(n // KC):
        dtc = dt[:, c * KC:(c + 1) * KC]
        t = jnp.maximum(a + dtc, b + ALPHA * dtc)
        p16 = jnp.exp2(t).astype(jnp.bfloat16) * mask_chunk(c)
        part = jnp.dot(p16, wh_ref[pl.ds(c * KC, KC), :],
                       preferred_element_type=jnp.float32)
        pv = part if pv is None else pv + part
    return pv[:, :512] / pv[:, 512:513]


def _attn1_kernel(adj_ref, wh_ref, s_ref, dt_ref, maxd_ref, wn_ref, asd_ref,
                  mask_ref, who_ref, so_ref, dto_ref, maxdo_ref):
    i = pl.program_id(0)
    masked = adj_ref[...] > 0.99
    mask_ref[...] = masked.astype(jnp.int8)

    def mask_chunk(c):
        return masked[:, c * KC:(c + 1) * KC].astype(jnp.bfloat16)

    h = _attn_out(mask_chunk, wh_ref, s_ref, dt_ref, maxd_ref)
    wh = jnp.dot(jnp.maximum(h, 0.0), wn_ref[...],
                 preferred_element_type=jnp.float32)
    _wh_outputs(i, wh, asd_ref, who_ref, so_ref, dto_ref, maxdo_ref)


def _attn2_kernel(mask_ref, wh_ref, s_ref, dt_ref, maxd_ref, wn_ref, asd_ref,
                  who_ref, so_ref, dto_ref, maxdo_ref):
    i = pl.program_id(0)
    def mask_chunk(c):
        return mask_ref[:, pl.ds(c * KC, KC)].astype(jnp.bfloat16)

    h = _attn_out(mask_chunk, wh_ref, s_ref, dt_ref, maxd_ref)
    wh = jnp.dot(jnp.maximum(h, 0.0), wn_ref[...],
                 preferred_element_type=jnp.float32)
    _wh_outputs(i, wh, asd_ref, who_ref, so_ref, dto_ref, maxdo_ref)


def _attn3_kernel(mask_ref, wh_ref, s_ref, dt_ref, maxd_ref, o_ref):
    def mask_chunk(c):
        return mask_ref[:, pl.ds(c * KC, KC)].astype(jnp.bfloat16)

    o_ref[...] = _attn_out(mask_chunk, wh_ref, s_ref, dt_ref, maxd_ref)


def _attn_specs(n, first):
    mat_dtype = jnp.float32 if first else jnp.bfloat16
    in_specs = [
        pl.BlockSpec((BM, n), lambda i: (i, 0)),
        pl.BlockSpec((n, NHA), lambda i: (0, 0)),
        pl.BlockSpec((BM, 1), lambda i: (i, 0)),
        pl.BlockSpec((1, n), lambda i: (0, 0)),
        pl.BlockSpec((1, 1), lambda i: (0, 0)),
    ]
    return mat_dtype, in_specs


def _attn_mid(mat, wh, s, dt, maxd, w_next, a_next, first):
    n = s.shape[0]
    _, in_specs = _attn_specs(n, first)
    in_specs += [
        pl.BlockSpec((512, 512), lambda i: (0, 0)),
        pl.BlockSpec((512, 2), lambda i: (0, 0)),
    ]
    out_specs, out_shape = _wh_specs(n)
    if first:
        body = _attn1_kernel
        out_specs = [pl.BlockSpec((BM, n), lambda i: (i, 0))] + out_specs
        out_shape = [jax.ShapeDtypeStruct((n, n), jnp.int8)] + out_shape
    else:
        body = _attn2_kernel
    return pl.pallas_call(
        body,
        grid=(n // BM,),
        in_specs=in_specs,
        out_specs=out_specs,
        out_shape=out_shape,
        compiler_params=pltpu.CompilerParams(
            dimension_semantics=("arbitrary",),
        ),
    )(mat, wh, s, dt, maxd, w_next, _asd(a_next))


def _attn_last(mask, wh, s, dt, maxd):
    n = s.shape[0]
    _, in_specs = _attn_specs(n, False)
    return pl.pallas_call(
        _attn3_kernel,
        grid=(n // BM,),
        in_specs=in_specs,
        out_specs=pl.BlockSpec((BM, 512), lambda i: (i, 0)),
        out_shape=jax.ShapeDtypeStruct((n, 512), jnp.float32),
        compiler_params=pltpu.CompilerParams(
            dimension_semantics=("parallel",),
        ),
    )(mask, wh, s, dt, maxd)


def kernel(features, adj_matrix, W1, a1, W2, a2, W3, a3):
    wh, s, dt, maxd = _mm(features, W1, a1)
    mask, wh, s, dt, maxd = _attn_mid(adj_matrix, wh, s, dt, maxd,
                                      W2, a2, first=True)
    wh, s, dt, maxd = _attn_mid(mask, wh, s, dt, maxd, W3, a3, first=False)
    return _attn_last(mask, wh, s, dt, maxd)
